# baseline (speedup 1.0000x reference)
import jax
import jax.numpy as jnp
from jax import lax
from jax.experimental import pallas as pl
from jax.experimental.pallas import tpu as pltpu
from jax.experimental.pallas import tpu_sc as plsc

_DIM = 32
_LANES = 16
_NUM_CORES = 2
_NUM_SUBCORES = 16
_NUM_WORKERS = _NUM_CORES * _NUM_SUBCORES
_CHUNK = 128
_VPC = _CHUNK // _LANES
_BLK = _DIM * _CHUNK

_NROWS = 1000000
_MAIN_ROWS = (_NROWS // _CHUNK) * _CHUNK
_NCOLS = _MAIN_ROWS // _CHUNK
_TAIL = _NROWS - _MAIN_ROWS
_UW = 128
_UCOLS = _UW // _CHUNK
_FLAT = _DIM * _NROWS
_FLAT_TOTAL = _FLAT + _DIM * _TAIL


def _detile_body(entt_hbm, relt_hbm, etail_hbm, rtail_hbm,
                 entf_hbm, relf_hbm,
                 stg0, stg1, rsem0, rsem1, wsem0, wsem1):
    wid = lax.axis_index("s") * _NUM_CORES + lax.axis_index("c")
    slots = ((stg0, rsem0, wsem0), (stg1, rsem1, wsem1))
    nunits = _NCOLS // _UCOLS
    nu = nunits // _NUM_WORKERS + 1

    def do_table(src_hbm, dst_hbm):
        def read_fire(k, slot):
            c = wid + k * _NUM_WORKERS
            stg, rsem, _ = slots[slot]

            @pl.when(c < nunits)
            def _():
                col = pl.multiple_of(c * _UW, _CHUNK)
                pltpu.async_copy(
                    src_hbm.at[pl.ds(0, _DIM), pl.ds(col, _UW)], stg, rsem)

        def scatter(k, slot):
            c = wid + k * _NUM_WORKERS
            stg, rsem, wsem = slots[slot]
            dummy_src = src_hbm.at[pl.ds(0, _DIM), pl.ds(0, _UW)]

            @pl.when(c < nunits)
            def _():
                pltpu.make_async_copy(dummy_src, stg, rsem).wait()
                for d in range(_DIM):
                    for q in range(_UCOLS):
                        pltpu.async_copy(
                            stg.at[d, pl.ds(q * _CHUNK, _CHUNK)],
                            dst_hbm.at[pl.ds(
                                d * _NROWS + c * _UW + q * _CHUNK, _CHUNK)],
                            wsem)

        def recycle(k, slot):
            c = wid + k * _NUM_WORKERS
            stg, rsem, wsem = slots[slot]
            dummy_src = src_hbm.at[pl.ds(0, _DIM), pl.ds(0, _UW)]

            @pl.when(c < nunits)
            def _():
                pltpu.make_async_copy(dummy_src, stg, wsem).wait()

            read_fire(k + 2, slot)

        read_fire(0, 0)
        read_fire(1, 1)

        def pair(k2, carry):
            scatter(2 * k2, 0)
            scatter(2 * k2 + 1, 1)
            recycle(2 * k2, 0)
            recycle(2 * k2 + 1, 1)
            return carry

        lax.fori_loop(0, (nu + 1) // 2, pair, 0)

    do_table(entt_hbm, entf_hbm)
    do_table(relt_hbm, relf_hbm)

    @pl.when(wid == 0)
    def _():
        pltpu.sync_copy(etail_hbm, stg0.at[pl.ds(0, _DIM), pl.ds(0, _CHUNK)])
        for d in range(_DIM):
            pltpu.sync_copy(stg0.at[d, pl.ds(0, _TAIL)],
                            entf_hbm.at[pl.ds(_FLAT + d * _TAIL, _TAIL)])

    @pl.when(wid == 1)
    def _():
        pltpu.sync_copy(rtail_hbm, stg0.at[pl.ds(0, _DIM), pl.ds(0, _CHUNK)])
        for d in range(_DIM):
            pltpu.sync_copy(stg0.at[d, pl.ds(0, _TAIL)],
                            relf_hbm.at[pl.ds(_FLAT + d * _TAIL, _TAIL)])


def _score_body(hidx_hbm, ridx_hbm, tidx_hbm, entf_hbm, relf_hbm, dummy_hbm,
                out_hbm,
                hidx_v, ridx_v, tidx_v,
                hix0, rix0, tix0, hix1, rix1, tix1,
                hbuf0, rbuf0, tbuf0, hbuf1, rbuf1, tbuf1,
                out_v, sem0, sem1):
    wid = lax.axis_index("s") * _NUM_CORES + lax.axis_index("c")
    nchunks = hidx_v.shape[0]
    n = nchunks * _CHUNK

    pltpu.sync_copy(hidx_hbm.at[wid], hidx_v)
    pltpu.sync_copy(ridx_hbm.at[wid], ridx_v)
    pltpu.sync_copy(tidx_hbm.at[wid], tidx_v)

    bufs = (
        (hix0, rix0, tix0, hbuf0, rbuf0, tbuf0, sem0),
        (hix1, rix1, tix1, hbuf1, rbuf1, tbuf1, sem1),
    )

    def offsets(vec, d):
        main = vec + d * _NROWS
        tail = vec + (_FLAT - _MAIN_ROWS + d * _TAIL)
        return jnp.where(vec < _MAIN_ROWS, main, tail)

    def build_and_fire(c, slot):
        hix, rix, tix, hbuf, rbuf, tbuf, sem = bufs[slot]
        vecs = []
        for v in range(_VPC):
            ds_v = pl.ds(v * _LANES, _LANES)
            vecs.append((hidx_v[c, ds_v], ridx_v[c, ds_v], tidx_v[c, ds_v]))

        def d_body(d, carry):
            base = d * _CHUNK
            for v in range(_VPC):
                ds_v = pl.ds(base + v * _LANES, _LANES)
                hb, rb, tb = vecs[v]
                hix[ds_v] = offsets(hb, d)
                rix[ds_v] = offsets(rb, d)
                tix[ds_v] = offsets(tb, d)
            return carry

        lax.fori_loop(0, _DIM, d_body, 0)
        pltpu.async_copy(entf_hbm.at[hix], hbuf, sem)
        pltpu.async_copy(relf_hbm.at[rix], rbuf, sem)
        pltpu.async_copy(entf_hbm.at[tix], tbuf, sem)

    def drain(slot):
        _, _, _, hbuf, rbuf, tbuf, sem = bufs[slot]
        pltpu.make_async_copy(dummy_hbm, hbuf, sem).wait()
        pltpu.make_async_copy(dummy_hbm, rbuf, sem).wait()
        pltpu.make_async_copy(dummy_hbm, tbuf, sem).wait()

    def reduce(c, slot):
        _, _, _, hbuf, rbuf, tbuf, _ = bufs[slot]

        def d_body(d, accs):
            base = d * _CHUNK
            out = []
            for v in range(_VPC):
                ds_v = pl.ds(base + v * _LANES, _LANES)
                a = jnp.abs(hbuf[ds_v] + rbuf[ds_v] - tbuf[ds_v])
                out.append(accs[v] + a)
            return tuple(out)

        zero = jnp.zeros((_LANES,), jnp.float32)
        accs = lax.fori_loop(0, _DIM, d_body, (zero,) * _VPC)
        for v in range(_VPC):
            out_v[pl.ds(c * _CHUNK + v * _LANES, _LANES)] = accs[v]

    build_and_fire(0, 0)
    build_and_fire(1, 1)

    def pair_body(p, carry):
        c0 = 2 * p
        drain(0)
        reduce(c0, 0)

        @pl.when(c0 + 2 < nchunks)
        def _():
            build_and_fire(c0 + 2, 0)

        drain(1)
        reduce(c0 + 1, 1)

        @pl.when(c0 + 3 < nchunks)
        def _():
            build_and_fire(c0 + 3, 1)

        return carry

    lax.fori_loop(0, nchunks // 2, pair_body, 0)

    pltpu.sync_copy(out_v, out_hbm.at[pl.ds(wid * n, n)])


def kernel(pos_triples, neg_triples, entity_emb, relation_emb):
    trip = jnp.concatenate([pos_triples, neg_triples], axis=0)
    total = trip.shape[0]
    n = total // _NUM_WORKERS
    nchunks = n // _CHUNK
    hidx = trip[:, 0].reshape(_NUM_WORKERS, nchunks, _CHUNK)
    ridx = trip[:, 1].reshape(_NUM_WORKERS, nchunks, _CHUNK)
    tidx = trip[:, 2].reshape(_NUM_WORKERS, nchunks, _CHUNK)

    entt = entity_emb.T
    relt = relation_emb.T
    etail = entity_emb[_MAIN_ROWS:, :].T
    rtail = relation_emb[_MAIN_ROWS:, :].T
    etail = jnp.pad(etail, ((0, 0), (0, _CHUNK - _TAIL)))
    rtail = jnp.pad(rtail, ((0, 0), (0, _CHUNK - _TAIL)))

    mesh = plsc.VectorSubcoreMesh(core_axis_name="c", subcore_axis_name="s")

    detile = pl.kernel(
        _detile_body,
        mesh=mesh,
        out_type=(
            jax.ShapeDtypeStruct((_FLAT_TOTAL,), jnp.float32),
            jax.ShapeDtypeStruct((_FLAT_TOTAL,), jnp.float32),
        ),
        scratch_types=[
            pltpu.VMEM((_DIM, _UW), jnp.float32),
            pltpu.VMEM((_DIM, _UW), jnp.float32),
            pltpu.SemaphoreType.DMA,
            pltpu.SemaphoreType.DMA,
            pltpu.SemaphoreType.DMA,
            pltpu.SemaphoreType.DMA,
        ],
    )
    entf, relf = detile(entt, relt, etail, rtail)

    dummy = jnp.zeros((_BLK,), jnp.float32)
    ix = pltpu.VMEM((_BLK,), jnp.int32)
    buf = pltpu.VMEM((_BLK,), jnp.float32)
    score = pl.kernel(
        _score_body,
        mesh=mesh,
        compiler_params=pltpu.CompilerParams(use_tc_tiling_on_sc=False),
        out_type=jax.ShapeDtypeStruct((total,), jnp.float32),
        scratch_types=[
            pltpu.VMEM((nchunks, _CHUNK), jnp.int32),
            pltpu.VMEM((nchunks, _CHUNK), jnp.int32),
            pltpu.VMEM((nchunks, _CHUNK), jnp.int32),
            ix, ix, ix, ix, ix, ix,
            buf, buf, buf, buf, buf, buf,
            pltpu.VMEM((n,), jnp.float32),
            pltpu.SemaphoreType.DMA,
            pltpu.SemaphoreType.DMA,
        ],
    )
    return score(hidx, ridx, tidx, entf, relf, dummy)


# speedup vs baseline: 1.0315x; 1.0315x over previous
import jax
import jax.numpy as jnp
from jax import lax
from jax.experimental import pallas as pl
from jax.experimental.pallas import tpu as pltpu
from jax.experimental.pallas import tpu_sc as plsc

_DIM = 32
_LANES = 16
_NUM_CORES = 2
_NUM_SUBCORES = 16
_NUM_WORKERS = _NUM_CORES * _NUM_SUBCORES
_CHUNK = 128
_VPC = _CHUNK // _LANES
_BLK = _DIM * _CHUNK

_NROWS = 1000000
_MAIN_ROWS = (_NROWS // _CHUNK) * _CHUNK
_NCOLS = _MAIN_ROWS // _CHUNK
_TAIL = _NROWS - _MAIN_ROWS
_UW = 128
_UCOLS = _UW // _CHUNK
_FLAT = _DIM * _NROWS
_FLAT_TOTAL = _FLAT + _DIM * _TAIL


def _detile_body(entt_hbm, relt_hbm, etail_hbm, rtail_hbm,
                 entf_hbm, relf_hbm,
                 stg0, stg1, rsem0, rsem1, wsem0, wsem1):
    wid = lax.axis_index("s") * _NUM_CORES + lax.axis_index("c")
    slots = ((stg0, rsem0, wsem0), (stg1, rsem1, wsem1))
    nunits = _NCOLS // _UCOLS
    nu = nunits // _NUM_WORKERS + 1

    def do_table(src_hbm, dst_hbm):
        def read_fire(k, slot):
            c = wid + k * _NUM_WORKERS
            stg, rsem, _ = slots[slot]

            @pl.when(c < nunits)
            def _():
                col = pl.multiple_of(c * _UW, _CHUNK)
                pltpu.async_copy(
                    src_hbm.at[pl.ds(0, _DIM), pl.ds(col, _UW)], stg, rsem)

        def unit(k, slot):
            c = wid + k * _NUM_WORKERS
            stg, rsem, wsem = slots[slot]
            dummy_src = src_hbm.at[pl.ds(0, _DIM), pl.ds(0, _UW)]

            @pl.when(c < nunits)
            def _():
                pltpu.make_async_copy(dummy_src, stg, rsem).wait()
                for d in range(_DIM):
                    for q in range(_UCOLS):
                        pltpu.async_copy(
                            stg.at[d, pl.ds(q * _CHUNK, _CHUNK)],
                            dst_hbm.at[pl.ds(
                                d * _NROWS + c * _UW + q * _CHUNK, _CHUNK)],
                            wsem)
                pltpu.make_async_copy(dummy_src, stg, wsem).wait()

            read_fire(k + 2, slot)

        read_fire(0, 0)
        read_fire(1, 1)

        def pair(k2, carry):
            unit(2 * k2, 0)
            unit(2 * k2 + 1, 1)
            return carry

        lax.fori_loop(0, (nu + 1) // 2, pair, 0)

    do_table(entt_hbm, entf_hbm)
    do_table(relt_hbm, relf_hbm)

    @pl.when(wid == 0)
    def _():
        pltpu.sync_copy(etail_hbm, stg0.at[pl.ds(0, _DIM), pl.ds(0, _CHUNK)])
        for d in range(_DIM):
            pltpu.sync_copy(stg0.at[d, pl.ds(0, _TAIL)],
                            entf_hbm.at[pl.ds(_FLAT + d * _TAIL, _TAIL)])

    @pl.when(wid == 1)
    def _():
        pltpu.sync_copy(rtail_hbm, stg0.at[pl.ds(0, _DIM), pl.ds(0, _CHUNK)])
        for d in range(_DIM):
            pltpu.sync_copy(stg0.at[d, pl.ds(0, _TAIL)],
                            relf_hbm.at[pl.ds(_FLAT + d * _TAIL, _TAIL)])


def _score_body(hidx_hbm, ridx_hbm, tidx_hbm, entf_hbm, relf_hbm, dummy_hbm,
                out_hbm,
                hidx_v, ridx_v, tidx_v,
                hix0, rix0, tix0, hix1, rix1, tix1,
                hbuf0, rbuf0, tbuf0, hbuf1, rbuf1, tbuf1,
                out_v, sem0, sem1):
    wid = lax.axis_index("s") * _NUM_CORES + lax.axis_index("c")
    nchunks = hidx_v.shape[0]
    n = nchunks * _CHUNK

    pltpu.sync_copy(hidx_hbm.at[wid], hidx_v)
    pltpu.sync_copy(ridx_hbm.at[wid], ridx_v)
    pltpu.sync_copy(tidx_hbm.at[wid], tidx_v)

    bufs = (
        (hix0, rix0, tix0, hbuf0, rbuf0, tbuf0, sem0),
        (hix1, rix1, tix1, hbuf1, rbuf1, tbuf1, sem1),
    )

    def offsets(vec, d):
        main = vec + d * _NROWS
        tail = vec + (_FLAT - _MAIN_ROWS + d * _TAIL)
        return jnp.where(vec < _MAIN_ROWS, main, tail)

    def build_and_fire(c, slot):
        hix, rix, tix, hbuf, rbuf, tbuf, sem = bufs[slot]
        vecs = []
        for v in range(_VPC):
            ds_v = pl.ds(v * _LANES, _LANES)
            vecs.append((hidx_v[c, ds_v], ridx_v[c, ds_v], tidx_v[c, ds_v]))

        def d_body(d, carry):
            base = d * _CHUNK
            for v in range(_VPC):
                ds_v = pl.ds(base + v * _LANES, _LANES)
                hb, rb, tb = vecs[v]
                hix[ds_v] = offsets(hb, d)
                rix[ds_v] = offsets(rb, d)
                tix[ds_v] = offsets(tb, d)
            return carry

        lax.fori_loop(0, _DIM, d_body, 0)
        pltpu.async_copy(entf_hbm.at[hix], hbuf, sem)
        pltpu.async_copy(relf_hbm.at[rix], rbuf, sem)
        pltpu.async_copy(entf_hbm.at[tix], tbuf, sem)

    def drain(slot):
        _, _, _, hbuf, rbuf, tbuf, sem = bufs[slot]
        pltpu.make_async_copy(dummy_hbm, hbuf, sem).wait()
        pltpu.make_async_copy(dummy_hbm, rbuf, sem).wait()
        pltpu.make_async_copy(dummy_hbm, tbuf, sem).wait()

    def reduce(c, slot):
        _, _, _, hbuf, rbuf, tbuf, _ = bufs[slot]

        def d_body(d, accs):
            base = d * _CHUNK
            out = []
            for v in range(_VPC):
                ds_v = pl.ds(base + v * _LANES, _LANES)
                a = jnp.abs(hbuf[ds_v] + rbuf[ds_v] - tbuf[ds_v])
                out.append(accs[v] + a)
            return tuple(out)

        zero = jnp.zeros((_LANES,), jnp.float32)
        accs = lax.fori_loop(0, _DIM, d_body, (zero,) * _VPC)
        for v in range(_VPC):
            out_v[pl.ds(c * _CHUNK + v * _LANES, _LANES)] = accs[v]

    build_and_fire(0, 0)
    build_and_fire(1, 1)

    def pair_body(p, carry):
        c0 = 2 * p
        drain(0)
        reduce(c0, 0)

        @pl.when(c0 + 2 < nchunks)
        def _():
            build_and_fire(c0 + 2, 0)

        drain(1)
        reduce(c0 + 1, 1)

        @pl.when(c0 + 3 < nchunks)
        def _():
            build_and_fire(c0 + 3, 1)

        return carry

    lax.fori_loop(0, nchunks // 2, pair_body, 0)

    pltpu.sync_copy(out_v, out_hbm.at[pl.ds(wid * n, n)])


def kernel(pos_triples, neg_triples, entity_emb, relation_emb):
    trip = jnp.concatenate([pos_triples, neg_triples], axis=0)
    total = trip.shape[0]
    n = total // _NUM_WORKERS
    nchunks = n // _CHUNK
    hidx = trip[:, 0].reshape(_NUM_WORKERS, nchunks, _CHUNK)
    ridx = trip[:, 1].reshape(_NUM_WORKERS, nchunks, _CHUNK)
    tidx = trip[:, 2].reshape(_NUM_WORKERS, nchunks, _CHUNK)

    entt = entity_emb.T
    relt = relation_emb.T
    etail = entity_emb[_MAIN_ROWS:, :].T
    rtail = relation_emb[_MAIN_ROWS:, :].T
    etail = jnp.pad(etail, ((0, 0), (0, _CHUNK - _TAIL)))
    rtail = jnp.pad(rtail, ((0, 0), (0, _CHUNK - _TAIL)))

    mesh = plsc.VectorSubcoreMesh(core_axis_name="c", subcore_axis_name="s")

    detile = pl.kernel(
        _detile_body,
        mesh=mesh,
        out_type=(
            jax.ShapeDtypeStruct((_FLAT_TOTAL,), jnp.float32),
            jax.ShapeDtypeStruct((_FLAT_TOTAL,), jnp.float32),
        ),
        scratch_types=[
            pltpu.VMEM((_DIM, _UW), jnp.float32),
            pltpu.VMEM((_DIM, _UW), jnp.float32),
            pltpu.SemaphoreType.DMA,
            pltpu.SemaphoreType.DMA,
            pltpu.SemaphoreType.DMA,
            pltpu.SemaphoreType.DMA,
        ],
    )
    entf, relf = detile(entt, relt, etail, rtail)

    dummy = jnp.zeros((_BLK,), jnp.float32)
    ix = pltpu.VMEM((_BLK,), jnp.int32)
    buf = pltpu.VMEM((_BLK,), jnp.float32)
    score = pl.kernel(
        _score_body,
        mesh=mesh,
        compiler_params=pltpu.CompilerParams(use_tc_tiling_on_sc=False),
        out_type=jax.ShapeDtypeStruct((total,), jnp.float32),
        scratch_types=[
            pltpu.VMEM((nchunks, _CHUNK), jnp.int32),
            pltpu.VMEM((nchunks, _CHUNK), jnp.int32),
            pltpu.VMEM((nchunks, _CHUNK), jnp.int32),
            ix, ix, ix, ix, ix, ix,
            buf, buf, buf, buf, buf, buf,
            pltpu.VMEM((n,), jnp.float32),
            pltpu.SemaphoreType.DMA,
            pltpu.SemaphoreType.DMA,
        ],
    )
    return score(hidx, ridx, tidx, entf, relf, dummy)
